# baseline (speedup 1.0000x reference)
import functools

import jax
import jax.numpy as jnp
from jax import lax
from jax.experimental import pallas as pl
from jax.experimental.pallas import tpu as pltpu
from jax.experimental.pallas import tpu_sc as plsc

N_ACC = 50000
N_MER = 10000
D = 128
H = 64
E_TXN = 600000
E_CHG = 200000

NS = 16
HALF = 25088
NROWS = 2 * HALF
RPT = HALF // NS
ZR = 112

K = 96
SB = 2
SBE = SB * K

NSB_TXN = E_TXN // SBE
E_CHG_PAD = -(-E_CHG // SBE) * SBE
NSB_CHG = E_CHG_PAD // SBE


def _seg_sum_sc(lin_txn, lin_chg, et_flat, ec_flat, vt, vc):
    mesh = plsc.VectorSubcoreMesh(core_axis_name="c", subcore_axis_name="s")

    @functools.partial(
        pl.kernel,
        mesh=mesh,
        compiler_params=pltpu.CompilerParams(use_tc_tiling_on_sc=False,
                                             needs_layout_passes=False),
        out_type=jax.ShapeDtypeStruct((NROWS, H), jnp.float32),
        scratch_types=[
            pltpu.VMEM((2, SBE), jnp.int32),
            pltpu.VMEM((2, SBE), jnp.int32),
            pltpu.VMEM((2, SBE), jnp.float32),
            pltpu.VMEM((2, SB, K), jnp.int32),
            pltpu.VMEM((2, SBE, H), jnp.float32),
            pltpu.VMEM_SHARED((HALF, H), jnp.float32),
            pltpu.SemaphoreType.DMA,
            pltpu.SemaphoreType.DMA,
            pltpu.SemaphoreType.DMA,
            pltpu.SemaphoreType.DMA,
            pltpu.SemaphoreType.DMA,
            pltpu.SemaphoreType.DMA,
        ],
    )
    def body(lin_t_hbm, lin_c_hbm, et_hbm, ec_hbm, vt_hbm, vc_hbm, out_hbm,
             di, si, vi, ldst, rows, acc,
             gsem0, gsem1, ssem0, ssem1, isem0, isem1):
        c = lax.axis_index("c")
        s = lax.axis_index("s")
        rbase = c * HALF
        gsem = (gsem0, gsem1)
        ssem = (ssem0, ssem1)
        isem = (isem0, isem1)

        def zfill(i, _):
            for q in range(H // 16):
                rows[0, i, pl.ds(q * 16, 16)] = jnp.zeros((16,), jnp.float32)
            return 0

        lax.fori_loop(0, ZR, zfill, 0)

        def zcopy(q, _):
            pltpu.sync_copy(rows.at[0, pl.ds(0, ZR)],
                            acc.at[pl.ds(s * RPT + q * ZR, ZR)])
            return 0

        lax.fori_loop(0, RPT // ZR, zcopy, 0)
        plsc.subcore_barrier()

        def make_rel(tab_hbm, e_hbm, v_hbm, nedge, nsb):
            def load_idx(st, m):
                sb = jnp.minimum(s + m * NS, nsb - 1)
                base = sb * SBE
                pltpu.async_copy(e_hbm.at[pl.ds(base, SBE)], di.at[st],
                                 isem[st])
                pltpu.async_copy(e_hbm.at[pl.ds(nedge + base, SBE)],
                                 si.at[st], isem[st])
                pltpu.async_copy(v_hbm.at[pl.ds(base, SBE)], vi.at[st],
                                 isem[st])

            def wait_idx(st):
                pltpu.make_async_copy(e_hbm.at[pl.ds(0, SBE)], di.at[st],
                                      isem[st]).wait()
                pltpu.make_async_copy(e_hbm.at[pl.ds(0, SBE)], si.at[st],
                                      isem[st]).wait()
                pltpu.make_async_copy(v_hbm.at[pl.ds(0, SBE)], vi.at[st],
                                      isem[st]).wait()

            def issue_gathers(st):
                for j in range(SB):
                    pltpu.async_copy(
                        tab_hbm.at[si.at[st, pl.ds(j * K, K)]],
                        rows.at[st, pl.ds(j * K, K)], gsem[st])

            def wait_gathers(st):
                for j in range(SB):
                    pltpu.make_async_copy(
                        tab_hbm.at[si.at[st, pl.ds(j * K, K)]],
                        rows.at[st, pl.ds(j * K, K)], gsem[st]).wait()

            def issue_scatters(st):
                for j in range(SB):
                    pltpu.async_copy(
                        rows.at[st, pl.ds(j * K, K)],
                        acc.at[ldst.at[st, j]], ssem[st], add=True)

            def wait_scatters(st):
                for j in range(SB):
                    pltpu.make_async_copy(
                        rows.at[st, pl.ds(j * K, K)],
                        acc.at[ldst.at[st, j]], ssem[st]).wait()

            def prep(st, first):
                wait_idx(st)
                if not first:
                    wait_scatters(st)
                issue_gathers(st)

            def process(st, m, mnext):
                okf = jnp.where(s + m * NS < nsb, 1.0, 0.0)
                wait_gathers(st)

                def scale_chunk(j, _):
                    def scale_vg(v, _):
                        off = j * K + v * 16
                        sl = pl.ds(off, 16)
                        d = di[st, sl]
                        ld = jnp.where(d >= HALF, d - HALF, d)
                        ldst[st, j, pl.ds(v * 16, 16)] = ld
                        mk = (d >= rbase) & (d < rbase + HALF)
                        vv = jnp.where(mk, vi[st, sl], 0.0) * okf
                        vi[st, sl] = vv
                        for kk in range(16):
                            fidx = jnp.full((16,), off + kk, jnp.int32)
                            f = plsc.load_gather(vi.at[st], [fidx])
                            for q in range(H // 16):
                                sq = pl.ds(q * 16, 16)
                                rows[st, off + kk, sq] = rows[st, off + kk, sq] * f
                        return 0

                    lax.fori_loop(0, K // 16, scale_vg, 0)
                    return 0

                lax.fori_loop(0, SB, scale_chunk, 0)
                issue_scatters(st)
                load_idx(st, mnext)

            trips = (nsb // NS - 1) // 2

            load_idx(0, 0)
            prep(0, True)
            load_idx(1, 1)
            prep(1, True)

            def step(i, _):
                process(0, 2 * i, 2 * i + 2)
                prep(0, False)
                process(1, 2 * i + 1, 2 * i + 3)
                prep(1, False)
                return 0

            lax.fori_loop(0, trips, step, 0)
            process(0, 2 * trips, 0)
            process(1, 2 * trips + 1, 0)
            wait_idx(0)
            wait_idx(1)
            wait_scatters(0)
            wait_scatters(1)

        make_rel(lin_t_hbm, et_hbm, vt_hbm, E_TXN, NSB_TXN)
        make_rel(lin_c_hbm, ec_hbm, vc_hbm, E_CHG_PAD, NSB_CHG)

        plsc.subcore_barrier()
        row0 = s * RPT
        pltpu.sync_copy(acc.at[pl.ds(row0, RPT)],
                        out_hbm.at[pl.ds(rbase + row0, RPT)])

    return body(lin_txn, lin_chg, et_flat, ec_flat, vt, vc)


def _mm_bias(x, w, b, block_rows, split=False):
    n, _ = x.shape
    _, dout = w.shape
    grid = n // block_rows

    def body(x_ref, w_ref, b_ref, *o_refs):
        y = (
            jnp.dot(x_ref[...], w_ref[...], preferred_element_type=jnp.float32)
            + b_ref[...]
        )
        if split:
            o_refs[0][...] = y[:, :H]
            o_refs[1][...] = y[:, H:]
        else:
            o_refs[0][...] = y

    if split:
        out_specs = [pl.BlockSpec((block_rows, H), lambda i: (i, 0))] * 2
        out_shape = [jax.ShapeDtypeStruct((n, H), jnp.float32)] * 2
    else:
        out_specs = pl.BlockSpec((block_rows, dout), lambda i: (i, 0))
        out_shape = jax.ShapeDtypeStruct((n, dout), jnp.float32)

    return pl.pallas_call(
        body,
        grid=(grid,),
        in_specs=[
            pl.BlockSpec((block_rows, x.shape[1]), lambda i: (i, 0)),
            pl.BlockSpec(w.shape, lambda i: (0, 0)),
            pl.BlockSpec((1, dout), lambda i: (0, 0)),
        ],
        out_specs=out_specs,
        out_shape=out_shape,
    )(x, w, b.reshape(1, dout))


def _head(p, neigh, w_out, b_out, block_rows):
    n = p.shape[0]
    grid = n // block_rows

    def body(p_ref, n_ref, w_ref, b_ref, o_ref):
        h = jnp.maximum(p_ref[...] + n_ref[...], 0.0)
        z = jnp.dot(h, w_ref[...], preferred_element_type=jnp.float32) + b_ref[...]
        o_ref[...] = jax.nn.sigmoid(z)

    return pl.pallas_call(
        body,
        grid=(grid,),
        in_specs=[
            pl.BlockSpec((block_rows, H), lambda i: (i, 0)),
            pl.BlockSpec((block_rows, H), lambda i: (i, 0)),
            pl.BlockSpec((H, 1), lambda i: (0, 0)),
            pl.BlockSpec((1, 1), lambda i: (0, 0)),
        ],
        out_specs=pl.BlockSpec((block_rows, 1), lambda i: (i, 0)),
        out_shape=jax.ShapeDtypeStruct((n, 1), jnp.float32),
    )(p, neigh, w_out, b_out.reshape(1, 1))


def kernel(feats_account, feats_merchant, edge_txn, edge_chg, val_txn, val_chg,
           W_proj_acc, b_proj_acc, W_proj_mer, b_proj_mer,
           W_rel_txn, b_rel_txn, W_rel_chg, b_rel_chg,
           W_comb_acc, b_comb_acc, W_comb_mer, b_comb_mer,
           W_out, b_out):
    Wc1 = W_comb_acc[:H]
    Wc2 = W_comb_acc[H:]
    Wf = W_proj_acc @ Wc1
    bf = b_proj_acc @ Wc1 + b_comb_acc
    Wcat = jnp.concatenate([W_rel_txn @ Wc2, Wf], axis=1)
    bcat = jnp.concatenate([b_rel_txn @ Wc2, bf])

    lin_txn, p_acc = _mm_bias(feats_account, Wcat, bcat, 2000, split=True)
    lin_chg = _mm_bias(feats_merchant, W_rel_chg @ Wc2, b_rel_chg @ Wc2, 2000)

    pad = E_CHG_PAD - E_CHG
    ec = jnp.pad(edge_chg, ((0, 0), (0, pad))).reshape(-1)
    vc = jnp.pad(val_chg, (0, pad))
    neigh = _seg_sum_sc(lin_txn, lin_chg, edge_txn.reshape(-1), ec,
                        val_txn, vc)

    out = _head(p_acc, neigh, W_out, b_out, 2000)
    return out[:, 0]


# speedup vs baseline: 2.3847x; 2.3847x over previous
import functools

import jax
import jax.numpy as jnp
from jax import lax
from jax.experimental import pallas as pl
from jax.experimental.pallas import tpu as pltpu
from jax.experimental.pallas import tpu_sc as plsc

N_ACC = 50000
N_MER = 10000
D = 128
H = 64
E_TXN = 600000
E_CHG = 200000

NS = 16
HALF = 25088
NROWS = 2 * HALF
RPT = HALF // NS
ZR = 112

K = 96
SB = 2
SBE = SB * K

NSB_TXN = E_TXN // SBE
E_CHG_PAD = -(-E_CHG // SBE) * SBE
NSB_CHG = E_CHG_PAD // SBE


def _seg_sum_sc(lin_txn, lin_chg, et_flat, ec_flat, vt, vc):
    mesh = plsc.VectorSubcoreMesh(core_axis_name="c", subcore_axis_name="s")

    @functools.partial(
        pl.kernel,
        mesh=mesh,
        compiler_params=pltpu.CompilerParams(use_tc_tiling_on_sc=False),
        out_type=jax.ShapeDtypeStruct((NROWS, H), jnp.float32),
        scratch_types=[
            pltpu.VMEM((2, SBE), jnp.int32),
            pltpu.VMEM((2, SBE), jnp.int32),
            pltpu.VMEM((2, SBE), jnp.float32),
            pltpu.VMEM((2, SB, K), jnp.int32),
            pltpu.VMEM((2, SBE, H), jnp.float32),
            pltpu.VMEM_SHARED((HALF, H), jnp.float32),
            pltpu.SemaphoreType.DMA,
            pltpu.SemaphoreType.DMA,
            pltpu.SemaphoreType.DMA,
            pltpu.SemaphoreType.DMA,
            pltpu.SemaphoreType.DMA,
            pltpu.SemaphoreType.DMA,
        ],
    )
    def body(lin_t_hbm, lin_c_hbm, et_hbm, ec_hbm, vt_hbm, vc_hbm, out_hbm,
             di, si, vi, ldst, rows, acc,
             gsem0, gsem1, ssem0, ssem1, isem0, isem1):
        c = lax.axis_index("c")
        s = lax.axis_index("s")
        rbase = c * HALF
        gsem = (gsem0, gsem1)
        ssem = (ssem0, ssem1)
        isem = (isem0, isem1)

        def zfill(i, _):
            for q in range(H // 16):
                rows[0, i, pl.ds(q * 16, 16)] = jnp.zeros((16,), jnp.float32)
            return 0

        lax.fori_loop(0, ZR, zfill, 0)

        def zcopy(q, _):
            pltpu.sync_copy(rows.at[0, pl.ds(0, ZR)],
                            acc.at[pl.ds(s * RPT + q * ZR, ZR)])
            return 0

        lax.fori_loop(0, RPT // ZR, zcopy, 0)
        plsc.subcore_barrier()

        def make_rel(tab_hbm, e_hbm, v_hbm, nedge, nsb):
            def load_idx(st, m):
                sb = jnp.minimum(s + m * NS, nsb - 1)
                base = sb * SBE
                pltpu.async_copy(e_hbm.at[pl.ds(base, SBE)], di.at[st],
                                 isem[st])
                pltpu.async_copy(e_hbm.at[pl.ds(nedge + base, SBE)],
                                 si.at[st], isem[st])
                pltpu.async_copy(v_hbm.at[pl.ds(base, SBE)], vi.at[st],
                                 isem[st])

            def wait_idx(st):
                pltpu.make_async_copy(e_hbm.at[pl.ds(0, SBE)], di.at[st],
                                      isem[st]).wait()
                pltpu.make_async_copy(e_hbm.at[pl.ds(0, SBE)], si.at[st],
                                      isem[st]).wait()
                pltpu.make_async_copy(v_hbm.at[pl.ds(0, SBE)], vi.at[st],
                                      isem[st]).wait()

            def issue_gathers(st):
                for j in range(SB):
                    pltpu.async_copy(
                        tab_hbm.at[si.at[st, pl.ds(j * K, K)]],
                        rows.at[st, pl.ds(j * K, K)], gsem[st])

            def wait_gathers(st):
                for j in range(SB):
                    pltpu.make_async_copy(
                        tab_hbm.at[si.at[st, pl.ds(j * K, K)]],
                        rows.at[st, pl.ds(j * K, K)], gsem[st]).wait()

            def issue_scatters(st):
                for j in range(SB):
                    pltpu.async_copy(
                        rows.at[st, pl.ds(j * K, K)],
                        acc.at[ldst.at[st, j]], ssem[st], add=True)

            def wait_scatters(st):
                for j in range(SB):
                    pltpu.make_async_copy(
                        rows.at[st, pl.ds(j * K, K)],
                        acc.at[ldst.at[st, j]], ssem[st]).wait()

            def prep(st, first):
                wait_idx(st)
                if not first:
                    wait_scatters(st)
                issue_gathers(st)

            def process(st, m, mnext):
                okf = jnp.where(s + m * NS < nsb, 1.0, 0.0)
                wait_gathers(st)

                def scale_chunk(j, _):
                    for v in range(K // 16):
                        off = j * K + v * 16
                        sl = pl.ds(off, 16)
                        d = di[st, sl]
                        ld = jnp.where(d >= HALF, d - HALF, d)
                        ldst[st, j, pl.ds(v * 16, 16)] = ld
                        mk = (d >= rbase) & (d < rbase + HALF)
                        vv = jnp.where(mk, vi[st, sl], 0.0) * okf
                        for kk in range(16):
                            f = vv[kk]
                            for q in range(H // 16):
                                sq = pl.ds(q * 16, 16)
                                rows[st, off + kk, sq] = rows[st, off + kk, sq] * f
                    return 0

                lax.fori_loop(0, SB, scale_chunk, 0)
                issue_scatters(st)
                load_idx(st, mnext)

            trips = (nsb // NS - 1) // 2

            load_idx(0, 0)
            prep(0, True)
            load_idx(1, 1)
            prep(1, True)

            def step(i, _):
                process(0, 2 * i, 2 * i + 2)
                prep(0, False)
                process(1, 2 * i + 1, 2 * i + 3)
                prep(1, False)
                return 0

            lax.fori_loop(0, trips, step, 0)
            process(0, 2 * trips, 0)
            process(1, 2 * trips + 1, 0)
            wait_idx(0)
            wait_idx(1)
            wait_scatters(0)
            wait_scatters(1)

        make_rel(lin_t_hbm, et_hbm, vt_hbm, E_TXN, NSB_TXN)
        make_rel(lin_c_hbm, ec_hbm, vc_hbm, E_CHG_PAD, NSB_CHG)

        plsc.subcore_barrier()
        row0 = s * RPT
        pltpu.sync_copy(acc.at[pl.ds(row0, RPT)],
                        out_hbm.at[pl.ds(rbase + row0, RPT)])

    return body(lin_txn, lin_chg, et_flat, ec_flat, vt, vc)


def _mm_bias(x, w, b, block_rows, split=False):
    n, _ = x.shape
    _, dout = w.shape
    grid = n // block_rows

    def body(x_ref, w_ref, b_ref, *o_refs):
        y = (
            jnp.dot(x_ref[...], w_ref[...], preferred_element_type=jnp.float32)
            + b_ref[...]
        )
        if split:
            o_refs[0][...] = y[:, :H]
            o_refs[1][...] = y[:, H:]
        else:
            o_refs[0][...] = y

    if split:
        out_specs = [pl.BlockSpec((block_rows, H), lambda i: (i, 0))] * 2
        out_shape = [jax.ShapeDtypeStruct((n, H), jnp.float32)] * 2
    else:
        out_specs = pl.BlockSpec((block_rows, dout), lambda i: (i, 0))
        out_shape = jax.ShapeDtypeStruct((n, dout), jnp.float32)

    return pl.pallas_call(
        body,
        grid=(grid,),
        in_specs=[
            pl.BlockSpec((block_rows, x.shape[1]), lambda i: (i, 0)),
            pl.BlockSpec(w.shape, lambda i: (0, 0)),
            pl.BlockSpec((1, dout), lambda i: (0, 0)),
        ],
        out_specs=out_specs,
        out_shape=out_shape,
    )(x, w, b.reshape(1, dout))


def _head(p, neigh, w_out, b_out, block_rows):
    n = p.shape[0]
    grid = n // block_rows

    def body(p_ref, n_ref, w_ref, b_ref, o_ref):
        h = jnp.maximum(p_ref[...] + n_ref[...], 0.0)
        z = jnp.dot(h, w_ref[...], preferred_element_type=jnp.float32) + b_ref[...]
        o_ref[...] = jax.nn.sigmoid(z)

    return pl.pallas_call(
        body,
        grid=(grid,),
        in_specs=[
            pl.BlockSpec((block_rows, H), lambda i: (i, 0)),
            pl.BlockSpec((block_rows, H), lambda i: (i, 0)),
            pl.BlockSpec((H, 1), lambda i: (0, 0)),
            pl.BlockSpec((1, 1), lambda i: (0, 0)),
        ],
        out_specs=pl.BlockSpec((block_rows, 1), lambda i: (i, 0)),
        out_shape=jax.ShapeDtypeStruct((n, 1), jnp.float32),
    )(p, neigh, w_out, b_out.reshape(1, 1))


def kernel(feats_account, feats_merchant, edge_txn, edge_chg, val_txn, val_chg,
           W_proj_acc, b_proj_acc, W_proj_mer, b_proj_mer,
           W_rel_txn, b_rel_txn, W_rel_chg, b_rel_chg,
           W_comb_acc, b_comb_acc, W_comb_mer, b_comb_mer,
           W_out, b_out):
    Wc1 = W_comb_acc[:H]
    Wc2 = W_comb_acc[H:]
    Wf = W_proj_acc @ Wc1
    bf = b_proj_acc @ Wc1 + b_comb_acc
    Wcat = jnp.concatenate([W_rel_txn @ Wc2, Wf], axis=1)
    bcat = jnp.concatenate([b_rel_txn @ Wc2, bf])

    lin_txn, p_acc = _mm_bias(feats_account, Wcat, bcat, 2000, split=True)
    lin_chg = _mm_bias(feats_merchant, W_rel_chg @ Wc2, b_rel_chg @ Wc2, 2000)

    pad = E_CHG_PAD - E_CHG
    ec = jnp.pad(edge_chg, ((0, 0), (0, pad))).reshape(-1)
    vc = jnp.pad(val_chg, (0, pad))
    neigh = _seg_sum_sc(lin_txn, lin_chg, edge_txn.reshape(-1), ec,
                        val_txn, vc)

    out = _head(p_acc, neigh, W_out, b_out, 2000)
    return out[:, 0]
